# baseline (speedup 1.0000x reference)
import functools

import jax
import jax.numpy as jnp
from jax import lax
from jax.experimental import pallas as pl
from jax.experimental.pallas import tpu as pltpu
from jax.experimental.pallas import tpu_sc as plsc

_VOCAB = 100000
_EMBED = 128
_HIDDEN = 128
_BATCH = 1024
_SEQ = 200

_NC = 2
_NS = 16
_NW = _NC * _NS
_CH = 128
_NBUF = 4

_CHUNKS = (24, 40, 40, 48, 48)
_TB = 8


def _make_sc_body(nch):
    def body(table_hbm, idx_hbm, out_hbm, idx_v,
             buf0, buf1, buf2, buf3, gs0, gs1, gs2, gs3,
             os0, os1, os2, os3):
        bufs = (buf0, buf1, buf2, buf3)
        gsems = (gs0, gs1, gs2, gs3)
        osems = (os0, os1, os2, os3)
        wid = lax.axis_index("s") * _NC + lax.axis_index("c")
        base = wid * (nch * _CH)
        pltpu.sync_copy(idx_hbm.at[wid], idx_v)

        def wb(j):
            return pltpu.async_copy(
                bufs[j % _NBUF],
                out_hbm.at[pl.ds(base + j * _CH, _CH)],
                osems[j % _NBUF])

        def wb_wait(j):
            pltpu.make_async_copy(
                bufs[j % _NBUF],
                out_hbm.at[pl.ds(base + j * _CH, _CH)],
                osems[j % _NBUF]).wait()

        def gather(j):
            return pltpu.async_copy(
                table_hbm.at[idx_v.at[j]], bufs[j % _NBUF],
                gsems[j % _NBUF])

        def gather_wait(j):
            pltpu.make_async_copy(
                table_hbm.at[idx_v.at[j]], bufs[j % _NBUF],
                gsems[j % _NBUF]).wait()

        for j in range(nch + 1):
            if j < nch:
                if j >= _NBUF:
                    wb_wait(j - _NBUF)
                gather(j)
            if j >= 1:
                gather_wait(j - 1)
                wb(j - 1)
        for j in range(max(0, nch - _NBUF + 1), nch):
            wb_wait(j)

    return body


@functools.cache
def _make_sc_gather(nch):
    mesh = plsc.VectorSubcoreMesh(core_axis_name="c", subcore_axis_name="s")
    return functools.partial(
        pl.kernel,
        mesh=mesh,
        out_type=jax.ShapeDtypeStruct((_NW * nch * _CH, _EMBED), jnp.float32),
        scratch_types=(
            [pltpu.VMEM((nch, _CH), jnp.int32)]
            + [pltpu.VMEM((_CH, _EMBED), jnp.float32) for _ in range(_NBUF)]
            + [pltpu.SemaphoreType.DMA for _ in range(2 * _NBUF)]
        ),
    )(_make_sc_body(nch))


def _gru_body(xlo_ref, xhi_ref, h0_ref, wih_ref, whh_ref, bsum_ref, bihn_ref,
              out_ref, h_ref):
    t = pl.program_id(0)

    @pl.when(t == 0)
    def _init():
        h_ref[...] = h0_ref[...]

    h = h_ref[...]
    wih = wih_ref[...]
    whh = whh_ref[...]
    bsum = bsum_ref[...]
    bihn = bihn_ref[...]
    H = _HIDDEN
    for k in range(_TB):
        xk = xlo_ref[k] if k < _TB // 2 else xhi_ref[k - _TB // 2]
        gi = jnp.dot(xk.astype(jnp.bfloat16), wih,
                     preferred_element_type=jnp.float32)
        gh = jnp.dot(h.astype(jnp.bfloat16), whh,
                     preferred_element_type=jnp.float32) + bsum
        r = 0.5 * jnp.tanh(0.5 * (gi[:, 0:H] + gh[:, 0:H])) + 0.5
        z = 0.5 * jnp.tanh(0.5 * (gi[:, H:2 * H] + gh[:, H:2 * H])) + 0.5
        n = jnp.tanh(gi[:, 2 * H:] + (r * gh[:, 2 * H:] + bihn))
        h = n + z * (h - n)
    h_ref[...] = h

    @pl.when(t == pl.num_programs(0) - 1)
    def _emit():
        out_ref[...] = h


@functools.cache
def _make_tc_gru(steps):
    return pl.pallas_call(
        _gru_body,
        grid=(steps // _TB,),
        in_specs=[
            pl.BlockSpec((_TB // 2, _BATCH, _EMBED), lambda t: (2 * t, 0, 0)),
            pl.BlockSpec((_TB // 2, _BATCH, _EMBED),
                         lambda t: (2 * t + 1, 0, 0)),
            pl.BlockSpec((_BATCH, _HIDDEN), lambda t: (0, 0)),
            pl.BlockSpec((_EMBED, 3 * _HIDDEN), lambda t: (0, 0)),
            pl.BlockSpec((_HIDDEN, 3 * _HIDDEN), lambda t: (0, 0)),
            pl.BlockSpec((1, 3 * _HIDDEN), lambda t: (0, 0)),
            pl.BlockSpec((1, _HIDDEN), lambda t: (0, 0)),
        ],
        out_specs=pl.BlockSpec((_BATCH, _HIDDEN), lambda t: (0, 0)),
        out_shape=jax.ShapeDtypeStruct((_BATCH, _HIDDEN), jnp.float32),
        scratch_shapes=[pltpu.VMEM((_BATCH, _HIDDEN), jnp.float32)],
    )


def kernel(source, table, W_ih, W_hh, b_ih, b_hh):
    idx_chunks = []
    flat = source.astype(jnp.int32).T.reshape(-1)
    off = 0
    for s in _CHUNKS:
        rows = s * _BATCH
        idx_chunks.append(flat[off:off + rows].reshape(_NW, rows // (_NW * _CH), _CH))
        off += rows

    wih_bf = W_ih.astype(jnp.bfloat16)
    whh_bf = W_hh.astype(jnp.bfloat16)
    bsum = jnp.concatenate([b_ih[:2 * _HIDDEN] + b_hh[:2 * _HIDDEN],
                            b_hh[2 * _HIDDEN:]]).reshape(1, -1)
    bihn = b_ih[2 * _HIDDEN:].reshape(1, -1)

    gathered = []
    for c, s in enumerate(_CHUNKS):
        nch = s * _BATCH // (_NW * _CH)
        gathered.append(_make_sc_gather(nch)(table, idx_chunks[c]))

    h = jnp.zeros((_BATCH, _HIDDEN), jnp.float32)
    for c, s in enumerate(_CHUNKS):
        xs = gathered[c].reshape(s, _BATCH, _EMBED)
        h = _make_tc_gru(s)(xs, xs, h, wih_bf, whh_bf, bsum, bihn)
    return h
